# baseline (speedup 1.0000x reference)
import functools
import jax
import jax.numpy as jnp
from jax import lax
from jax.experimental import pallas as pl
from jax.experimental.pallas import tpu as pltpu
from jax.experimental.pallas import tpu_sc as plsc

_T = 16
_L = 16


def _make_sc_kernel(BS, S, D, n_workers, rows_per_worker):
    mesh = plsc.VectorSubcoreMesh(core_axis_name="c", subcore_axis_name="s")
    n_chunks = rows_per_worker // _T

    @functools.partial(
        pl.kernel,
        mesh=mesh,
        out_type=jax.ShapeDtypeStruct((BS, D), jnp.float32),
        scratch_types=[
            pltpu.VMEM((_T, D), jnp.float32),
            pltpu.VMEM((_T, D), jnp.float32),
            pltpu.VMEM((_T, D), jnp.float32),
            pltpu.VMEM((_T,), jnp.int32),
            pltpu.SemaphoreType.DMA,
        ],
    )
    def sc_k(x_hbm, lab_hbm, seg_hbm, pos_hbm, out_hbm, x_v, pos_v, seg_v, idx_v, sem):
        wid = lax.axis_index("s") * 2 + lax.axis_index("c")
        base = wid * rows_per_worker
        pos_base = base % S

        def chunk_body(i, carry):
            tok = base + i * _T
            pltpu.sync_copy(lab_hbm.at[pl.ds(tok, _T)], idx_v)
            cp = pltpu.async_copy(seg_hbm.at[idx_v], seg_v, sem)
            pltpu.sync_copy(x_hbm.at[pl.ds(tok, _T)], x_v)
            pltpu.sync_copy(pos_hbm.at[pl.ds(pos_base + i * _T, _T)], pos_v)
            cp.wait()

            def col_body(ci, c2):
                sl = pl.ds(ci * _L, _L)
                for r in range(_T):
                    plsc.addupdate(x_v.at[r, sl], pos_v[r, sl] + seg_v[r, sl])
                return c2

            lax.fori_loop(0, D // _L, col_body, 0)
            pltpu.sync_copy(x_v, out_hbm.at[pl.ds(tok, _T)])
            return carry

        lax.fori_loop(0, n_chunks, chunk_body, 0)

    return sc_k


def kernel(x, segment_label, seg_table, pos_table):
    B, S, D = x.shape
    BS = B * S
    n_workers = 32
    rows_per_worker = BS // n_workers

    x2 = x.reshape(BS, D)
    lab = segment_label.astype(jnp.int32).reshape(BS)

    sc_k = _make_sc_kernel(BS, S, D, n_workers, rows_per_worker)
    out = sc_k(x2, lab, seg_table, pos_table)
    return out.reshape(B, S, D)


# speedup vs baseline: 1.7285x; 1.7285x over previous
import functools
import jax
import jax.numpy as jnp
from jax import lax
from jax.experimental import pallas as pl
from jax.experimental.pallas import tpu as pltpu
from jax.experimental.pallas import tpu_sc as plsc

_T = 32
_L = 16


def _make_sc_kernel(BS, S, D, n_workers, rows_per_worker):
    mesh = plsc.VectorSubcoreMesh(core_axis_name="c", subcore_axis_name="s")
    n_chunks = rows_per_worker // _T

    @functools.partial(
        pl.kernel,
        mesh=mesh,
        out_type=jax.ShapeDtypeStruct((BS, D), jnp.float32),
        scratch_types=[
            pltpu.VMEM((_T, D), jnp.float32),
            pltpu.VMEM((_T, D), jnp.float32),
            pltpu.VMEM((3 * D,), jnp.float32),
            pltpu.VMEM((_T,), jnp.int32),
        ],
    )
    def sc_k(x_hbm, lab_hbm, seg_hbm, pos_hbm, out_hbm, x_v, pos_v, tab_v, idx_v):
        wid = lax.axis_index("s") * 2 + lax.axis_index("c")
        base = wid * rows_per_worker
        pos_base = base % S
        lanes = lax.iota(jnp.int32, _L)
        pltpu.sync_copy(seg_hbm, tab_v)

        def chunk_body(i, carry):
            tok = base + i * _T
            pltpu.sync_copy(lab_hbm.at[pl.ds(tok, _T)], idx_v)
            pltpu.sync_copy(x_hbm.at[pl.ds(tok, _T)], x_v)
            pltpu.sync_copy(pos_hbm.at[pl.ds(pos_base + i * _T, _T)], pos_v)

            for r0 in range(0, _T, _L):
                labv = idx_v[pl.ds(r0, _L)].astype(jnp.float32)
                splats = [
                    lax.gather(
                        labv,
                        jnp.full((_L, 1), r, dtype=jnp.int32),
                        dimension_numbers=lax.GatherDimensionNumbers(
                            offset_dims=(), collapsed_slice_dims=(0,),
                            start_index_map=(0,)),
                        slice_sizes=(1,),
                        mode=lax.GatherScatterMode.PROMISE_IN_BOUNDS,
                    )
                    for r in range(_L)
                ]
                w1 = [s * (2.0 - s) for s in splats]
                w2 = [s * (s - 1.0) * 0.5 for s in splats]

                def col_body(ci, c2, r0=r0, w1=w1, w2=w2):
                    sl = pl.ds(ci * _L, _L)
                    row0 = tab_v[pl.ds(ci * _L, _L)]
                    row1 = tab_v[pl.ds(D + ci * _L, _L)]
                    row2 = tab_v[pl.ds(2 * D + ci * _L, _L)]
                    d1 = row1 - row0
                    d2 = row2 - row0
                    for r in range(_L):
                        seg = row0 + w1[r] * d1 + w2[r] * d2
                        plsc.addupdate(x_v.at[r0 + r, sl], pos_v[r0 + r, sl] + seg)
                    return c2

                lax.fori_loop(0, D // _L, col_body, 0)
            pltpu.sync_copy(x_v, out_hbm.at[pl.ds(tok, _T)])
            return carry

        lax.fori_loop(0, n_chunks, chunk_body, 0)

    return sc_k


def kernel(x, segment_label, seg_table, pos_table):
    B, S, D = x.shape
    BS = B * S
    n_workers = 32
    rows_per_worker = BS // n_workers

    x2 = x.reshape(BS, D)
    lab = segment_label.astype(jnp.int32).reshape(BS)
    seg_flat = seg_table.reshape(3 * D)

    sc_k = _make_sc_kernel(BS, S, D, n_workers, rows_per_worker)
    out = sc_k(x2, lab, seg_flat, pos_table)
    return out.reshape(B, S, D)


# speedup vs baseline: 2.2053x; 1.2758x over previous
import functools
import jax
import jax.numpy as jnp
from jax import lax
from jax.experimental import pallas as pl
from jax.experimental.pallas import tpu as pltpu
from jax.experimental.pallas import tpu_sc as plsc

_T = 16
_L = 16
_NX = 4
_NP = 3


def _make_sc_kernel(BS, S, D, n_workers, rows_per_worker):
    mesh = plsc.VectorSubcoreMesh(core_axis_name="c", subcore_axis_name="s")
    n_chunks = rows_per_worker // _T

    scratch = (
        [pltpu.VMEM((_T, D), jnp.float32) for _ in range(_NX)]
        + [pltpu.VMEM((_T, D), jnp.float32) for _ in range(_NP)]
        + [pltpu.VMEM((3 * D,), jnp.float32),
           pltpu.VMEM((rows_per_worker,), jnp.int32)]
        + [pltpu.SemaphoreType.DMA for _ in range(_NX + _NX + _NP)]
    )

    @functools.partial(
        pl.kernel,
        mesh=mesh,
        out_type=jax.ShapeDtypeStruct((BS, D), jnp.float32),
        scratch_types=scratch,
    )
    def sc_k(x_hbm, lab_hbm, seg_hbm, pos_hbm, out_hbm, *bufs):
        xb = bufs[:_NX]
        pb = bufs[_NX:_NX + _NP]
        tab_v = bufs[_NX + _NP]
        idx_all = bufs[_NX + _NP + 1]
        sems = bufs[_NX + _NP + 2:]
        s_in = sems[:_NX]
        s_out = sems[_NX:2 * _NX]
        s_pos = sems[2 * _NX:]

        wid = lax.axis_index("s") * 2 + lax.axis_index("c")
        base = wid * rows_per_worker
        pos_base = base % S
        pltpu.sync_copy(seg_hbm, tab_v)
        pltpu.sync_copy(lab_hbm.at[pl.ds(base, rows_per_worker)], idx_all)

        pend_in = {}
        pend_out = {}

        def start_in(i):
            tok = base + i * _T
            cx = pltpu.async_copy(x_hbm.at[pl.ds(tok, _T)], xb[i % _NX], s_in[i % _NX])
            cp = pltpu.async_copy(
                pos_hbm.at[pl.ds(pos_base + i * _T, _T)], pb[i % _NP], s_pos[i % _NP])
            pend_in[i] = (cx, cp)

        def compute(i):
            x_v = xb[i % _NX]
            pos_v = pb[i % _NP]
            labv = idx_all[pl.ds(i * _T, _L)].astype(jnp.float32)
            splats = [
                lax.gather(
                    labv,
                    jnp.full((_L, 1), r, dtype=jnp.int32),
                    dimension_numbers=lax.GatherDimensionNumbers(
                        offset_dims=(), collapsed_slice_dims=(0,),
                        start_index_map=(0,)),
                    slice_sizes=(1,),
                    mode=lax.GatherScatterMode.PROMISE_IN_BOUNDS,
                )
                for r in range(_L)
            ]
            w1 = [s * (2.0 - s) for s in splats]
            w2 = [s * (s - 1.0) * 0.5 for s in splats]

            def col_body(ci, c2):
                sl = pl.ds(ci * _L, _L)
                row0 = tab_v[pl.ds(ci * _L, _L)]
                row1 = tab_v[pl.ds(D + ci * _L, _L)]
                row2 = tab_v[pl.ds(2 * D + ci * _L, _L)]
                d1 = row1 - row0
                d2 = row2 - row0
                for r in range(_L):
                    seg = row0 + w1[r] * d1 + w2[r] * d2
                    plsc.addupdate(x_v.at[r, sl], pos_v[r, sl] + seg)
                return c2

            lax.fori_loop(0, D // _L, col_body, 0)

        start_in(0)
        start_in(1)
        for i in range(n_chunks):
            if i + 2 < n_chunks:
                j = i - 2
                if j >= 0 and j in pend_out:
                    pend_out.pop(j).wait()
                start_in(i + 2)
            cx, cp = pend_in.pop(i)
            cx.wait()
            cp.wait()
            compute(i)
            pend_out[i] = pltpu.async_copy(
                xb[i % _NX], out_hbm.at[pl.ds(base + i * _T, _T)], s_out[i % _NX])
        for j in sorted(pend_out):
            pend_out[j].wait()

    return sc_k


def kernel(x, segment_label, seg_table, pos_table):
    B, S, D = x.shape
    BS = B * S
    n_workers = 32
    rows_per_worker = BS // n_workers

    x2 = x.reshape(BS, D)
    lab = segment_label.astype(jnp.int32).reshape(BS)
    seg_flat = seg_table.reshape(3 * D)

    sc_k = _make_sc_kernel(BS, S, D, n_workers, rows_per_worker)
    out = sc_k(x2, lab, seg_flat, pos_table)
    return out.reshape(B, S, D)
